# baseline (speedup 1.0000x reference)
import functools

import jax
import jax.numpy as jnp
from jax import lax
from jax.experimental import pallas as pl
from jax.experimental.pallas import tpu as pltpu
from jax.experimental.pallas import tpu_sc as plsc

_WEIGHT_DECAY = 1e-4
_D = 64
_B = 16384
_NC, _NS = 2, 16
_NW = _NC * _NS
_BW = _B // _NW
_CH = 128
_NCHUNK = _BW // _CH
_NGROUP = _BW // 16

_mesh = plsc.VectorSubcoreMesh(
    core_axis_name="c", subcore_axis_name="s",
    num_cores=_NC, num_subcores=_NS)


@functools.partial(
    pl.kernel,
    out_type=[
        jax.ShapeDtypeStruct((128, 128), jnp.float32),
        jax.ShapeDtypeStruct((_NW, 16), jnp.float32),
    ],
    mesh=_mesh,
    compiler_params=pltpu.CompilerParams(
        needs_layout_passes=False, use_tc_tiling_on_sc=False),
    scratch_types=[
        pltpu.VMEM((_NCHUNK, _CH), jnp.int32),
        pltpu.VMEM((_NCHUNK, _CH), jnp.int32),
        pltpu.VMEM((_NCHUNK, _CH), jnp.int32),
        pltpu.VMEM((_BW, _D), jnp.float32),
        pltpu.VMEM((_BW, _D), jnp.float32),
        pltpu.VMEM((_BW, _D), jnp.float32),
        pltpu.VMEM((_NCHUNK, _CH), jnp.float32),
        pltpu.VMEM((16,), jnp.float32),
        pltpu.SemaphoreType.DMA,
    ],
)
def _sc_bpr(emb_u, emb_i, raw_u, raw_i, users, pos, neg,
            out_diff, out_l2,
            idx_u, idx_p, idx_n, bu, bp, bn, dv, l2v, sem):
    wid = lax.axis_index("s") * _NC + lax.axis_index("c")
    row0 = wid * _NCHUNK

    pltpu.sync_copy(users.at[pl.ds(row0, _NCHUNK)], idx_u)
    pltpu.sync_copy(pos.at[pl.ds(row0, _NCHUNK)], idx_p)
    pltpu.sync_copy(neg.at[pl.ds(row0, _NCHUNK)], idx_n)

    def gather_all(tab_u, tab_i):
        copies = []
        for j in range(_NCHUNK):
            dst = pl.ds(j * _CH, _CH)
            copies.append(pltpu.async_copy(tab_u.at[idx_u.at[j]], bu.at[dst], sem))
            copies.append(pltpu.async_copy(tab_i.at[idx_p.at[j]], bp.at[dst], sem))
            copies.append(pltpu.async_copy(tab_i.at[idx_n.at[j]], bn.at[dst], sem))
        for c in copies:
            c.wait()

    gather_all(emb_u, emb_i)

    iota = jnp.arange(16, dtype=jnp.int32)

    def group_body(g, _):
        chunk = g // (_CH // 16)
        r = (g % (_CH // 16)) * 16
        i1 = g * 16 + iota
        acc = jnp.zeros((16,), jnp.float32)
        for d in range(_D):
            i2 = jnp.full((16,), d, jnp.int32)
            u = plsc.load_gather(bu, [i1, i2])
            p = plsc.load_gather(bp, [i1, i2])
            n = plsc.load_gather(bn, [i1, i2])
            acc = acc + u * (n - p)
        dv[chunk, pl.ds(r, 16)] = acc
        return 0

    lax.fori_loop(0, _NGROUP, group_body, 0)
    pltpu.sync_copy(dv, out_diff.at[pl.ds(row0, _NCHUNK)])

    gather_all(raw_u, raw_i)

    def l2_body(r, acc):
        for k in range(_D // 16):
            for buf in (bu, bp, bn):
                v = buf[r, pl.ds(k * 16, 16)]
                acc = acc + v * v
        return acc

    acc2 = lax.fori_loop(0, _BW, l2_body, jnp.zeros((16,), jnp.float32))
    l2v[...] = acc2
    pltpu.sync_copy(l2v, out_l2.at[wid])


def _combine_body(diff_ref, l2_ref, out_ref):
    x = diff_ref[...]
    sp = jnp.maximum(x, 0.0) + jnp.log1p(jnp.exp(-jnp.abs(x)))
    out_ref[0, 0] = (jnp.sum(sp) / _B
                     + (_WEIGHT_DECAY * 0.5 / _B) * jnp.sum(l2_ref[...]))


_combine = pl.pallas_call(
    _combine_body,
    out_shape=jax.ShapeDtypeStruct((1, 1), jnp.float32),
    out_specs=pl.BlockSpec(memory_space=pltpu.SMEM),
)


def kernel(emb_users, emb_items, raw_emb_users, raw_emb_items,
           users, pos_items, neg_items):
    u2 = users.reshape(128, 128)
    p2 = pos_items.reshape(128, 128)
    n2 = neg_items.reshape(128, 128)
    diff, l2 = _sc_bpr(emb_users, emb_items, raw_emb_users, raw_emb_items,
                       u2, p2, n2)
    out = _combine(diff, l2.reshape(4, 128))
    return out[0, 0]


# speedup vs baseline: 1.0131x; 1.0131x over previous
import functools

import jax
import jax.numpy as jnp
from jax import lax
from jax.experimental import pallas as pl
from jax.experimental.pallas import tpu as pltpu
from jax.experimental.pallas import tpu_sc as plsc

_WEIGHT_DECAY = 1e-4
_D = 64
_B = 16384
_NC, _NS = 2, 16
_NW = _NC * _NS
_BW = _B // _NW
_CH = 128
_NCHUNK = _BW // _CH
_NSLOT = 4

_mesh = plsc.VectorSubcoreMesh(
    core_axis_name="c", subcore_axis_name="s",
    num_cores=_NC, num_subcores=_NS)

_scratch = (
    [pltpu.VMEM((_NCHUNK, _CH), jnp.int32) for _ in range(3)]
    + [pltpu.VMEM((_CH, _D), jnp.float32) for _ in range(3 * _NSLOT)]
    + [pltpu.VMEM((_NCHUNK, _CH), jnp.float32),
       pltpu.VMEM((16,), jnp.float32)]
    + [pltpu.SemaphoreType.DMA for _ in range(_NSLOT)]
)


@functools.partial(
    pl.kernel,
    out_type=[
        jax.ShapeDtypeStruct((128, 128), jnp.float32),
        jax.ShapeDtypeStruct((_NW, 16), jnp.float32),
    ],
    mesh=_mesh,
    compiler_params=pltpu.CompilerParams(
        needs_layout_passes=False, use_tc_tiling_on_sc=False),
    scratch_types=_scratch,
)
def _sc_bpr(emb_u, emb_i, raw_u, raw_i, users, pos, neg,
            out_diff, out_l2, *refs):
    idx = refs[0:3]
    bufs = [refs[3 + 3 * s: 6 + 3 * s] for s in range(_NSLOT)]
    dv, l2v = refs[3 + 3 * _NSLOT], refs[4 + 3 * _NSLOT]
    sems = refs[5 + 3 * _NSLOT:]

    wid = lax.axis_index("s") * _NC + lax.axis_index("c")
    row0 = wid * _NCHUNK

    pltpu.sync_copy(users.at[pl.ds(row0, _NCHUNK)], idx[0])
    pltpu.sync_copy(pos.at[pl.ds(row0, _NCHUNK)], idx[1])
    pltpu.sync_copy(neg.at[pl.ds(row0, _NCHUNK)], idx[2])

    iota = jnp.arange(16, dtype=jnp.int32)
    ntask = 2 * _NCHUNK
    handles = [None] * ntask

    def issue(t):
        ph, ch = divmod(t, _NCHUNK)
        slot = t % _NSLOT
        tabs = (emb_u, emb_i, emb_i) if ph == 0 else (raw_u, raw_i, raw_i)
        handles[t] = [
            pltpu.async_copy(tabs[k].at[idx[k].at[ch]], bufs[slot][k], sems[slot])
            for k in range(3)
        ]

    def compute_cur(t):
        ch = t % _NCHUNK
        bu, bp, bn = bufs[t % _NSLOT]

        def group_body(g, _):
            i1 = g * 16 + iota
            accs = [jnp.zeros((16,), jnp.float32) for _ in range(4)]
            for d in range(_D):
                i2 = jnp.full((16,), d, jnp.int32)
                u = plsc.load_gather(bu, [i1, i2])
                p = plsc.load_gather(bp, [i1, i2])
                n = plsc.load_gather(bn, [i1, i2])
                accs[d % 4] = accs[d % 4] + u * (n - p)
            dv[ch, pl.ds(g * 16, 16)] = (accs[0] + accs[1]) + (accs[2] + accs[3])
            return 0

        lax.fori_loop(0, _CH // 16, group_body, 0)

    def compute_raw(t, l2accs):
        bu, bp, bn = bufs[t % _NSLOT]

        def rows_body(i, accs):
            a = list(accs)
            r = i * 4
            for rr in range(4):
                for k in range(_D // 16):
                    for bi, buf in enumerate((bu, bp, bn)):
                        v = buf[r + rr, pl.ds(k * 16, 16)]
                        a[(k + bi) % 4] = a[(k + bi) % 4] + v * v
            return tuple(a)

        return lax.fori_loop(0, _CH // 4, rows_body, l2accs)

    for t in range(3):
        issue(t)
    l2accs = tuple(jnp.zeros((16,), jnp.float32) for _ in range(4))
    for t in range(ntask):
        if t + 3 < ntask:
            issue(t + 3)
        for h in handles[t]:
            h.wait()
        if t < _NCHUNK:
            compute_cur(t)
        else:
            l2accs = compute_raw(t, l2accs)

    pltpu.sync_copy(dv, out_diff.at[pl.ds(row0, _NCHUNK)])
    l2v[...] = (l2accs[0] + l2accs[1]) + (l2accs[2] + l2accs[3])
    pltpu.sync_copy(l2v, out_l2.at[wid])


def _combine_body(diff_ref, l2_ref, out_ref):
    x = diff_ref[...]
    sp = jnp.maximum(x, 0.0) + jnp.log1p(jnp.exp(-jnp.abs(x)))
    out_ref[0, 0] = (jnp.sum(sp) / _B
                     + (_WEIGHT_DECAY * 0.5 / _B) * jnp.sum(l2_ref[...]))


_combine = pl.pallas_call(
    _combine_body,
    out_shape=jax.ShapeDtypeStruct((1, 1), jnp.float32),
    out_specs=pl.BlockSpec(memory_space=pltpu.SMEM),
)


def kernel(emb_users, emb_items, raw_emb_users, raw_emb_items,
           users, pos_items, neg_items):
    u2 = users.reshape(128, 128)
    p2 = pos_items.reshape(128, 128)
    n2 = neg_items.reshape(128, 128)
    diff, l2 = _sc_bpr(emb_users, emb_items, raw_emb_users, raw_emb_items,
                       u2, p2, n2)
    out = _combine(diff, l2.reshape(4, 128))
    return out[0, 0]


# speedup vs baseline: 1.1547x; 1.1397x over previous
import functools

import jax
import jax.numpy as jnp
from jax import lax
from jax.experimental import pallas as pl
from jax.experimental.pallas import tpu as pltpu
from jax.experimental.pallas import tpu_sc as plsc

_WEIGHT_DECAY = 1e-4
_D = 64
_B = 16384
_NC, _NS = 2, 16
_NW = _NC * _NS
_BW = _B // _NW
_CH = 128
_NCHUNK = _BW // _CH
_NSLOT = 4

_mesh = plsc.VectorSubcoreMesh(
    core_axis_name="c", subcore_axis_name="s",
    num_cores=_NC, num_subcores=_NS)

_scratch = (
    [pltpu.VMEM((_NCHUNK, _CH), jnp.int32) for _ in range(3)]
    + [pltpu.VMEM((_CH, _D), jnp.float32) for _ in range(3 * _NSLOT)]
    + [pltpu.VMEM((_NCHUNK, _CH), jnp.float32),
       pltpu.VMEM((16,), jnp.float32)]
    + [pltpu.SemaphoreType.DMA for _ in range(_NSLOT)]
)


@functools.partial(
    pl.kernel,
    out_type=[
        jax.ShapeDtypeStruct((128, 128), jnp.float32),
        jax.ShapeDtypeStruct((_NW, 16), jnp.float32),
    ],
    mesh=_mesh,
    compiler_params=pltpu.CompilerParams(
        needs_layout_passes=False, use_tc_tiling_on_sc=False),
    scratch_types=_scratch,
)
def _sc_bpr(emb_u, emb_i, raw_u, raw_i, users, pos, neg,
            out_diff, out_l2, *refs):
    idx = refs[0:3]
    bufs = [refs[3 + 3 * s: 6 + 3 * s] for s in range(_NSLOT)]
    dv, l2v = refs[3 + 3 * _NSLOT], refs[4 + 3 * _NSLOT]
    sems = refs[5 + 3 * _NSLOT:]

    wid = lax.axis_index("s") * _NC + lax.axis_index("c")
    row0 = wid * _NCHUNK

    pltpu.sync_copy(users.at[pl.ds(row0, _NCHUNK)], idx[0])
    pltpu.sync_copy(pos.at[pl.ds(row0, _NCHUNK)], idx[1])
    pltpu.sync_copy(neg.at[pl.ds(row0, _NCHUNK)], idx[2])

    iota = jnp.arange(16, dtype=jnp.int32)
    ntask = 2 * _NCHUNK
    handles = [None] * ntask

    def issue(t):
        ph, ch = divmod(t, _NCHUNK)
        slot = t % _NSLOT
        tabs = (emb_u, emb_i, emb_i) if ph == 0 else (raw_u, raw_i, raw_i)
        handles[t] = [
            pltpu.async_copy(tabs[k].at[idx[k].at[ch]], bufs[slot][k], sems[slot])
            for k in range(3)
        ]

    def compute_cur(t):
        ch = t % _NCHUNK
        bu, bp, bn = bufs[t % _NSLOT]

        def group_body(g, _):
            i1 = g * 16 + iota
            accs = [jnp.zeros((16,), jnp.float32) for _ in range(4)]
            for d in range(_D):
                i2 = iota + d
                i2 = i2 - jnp.where(i2 >= _D, _D, 0).astype(jnp.int32)
                u = plsc.load_gather(bu, [i1, i2])
                p = plsc.load_gather(bp, [i1, i2])
                n = plsc.load_gather(bn, [i1, i2])
                accs[d % 4] = accs[d % 4] + u * (n - p)
            dv[ch, pl.ds(g * 16, 16)] = (accs[0] + accs[1]) + (accs[2] + accs[3])
            return 0

        lax.fori_loop(0, _CH // 16, group_body, 0)

    def compute_raw(t, l2accs):
        bu, bp, bn = bufs[t % _NSLOT]

        def rows_body(i, accs):
            a = list(accs)
            r = i * 4
            for rr in range(4):
                for k in range(_D // 16):
                    for bi, buf in enumerate((bu, bp, bn)):
                        v = buf[r + rr, pl.ds(k * 16, 16)]
                        a[(k + bi) % 4] = a[(k + bi) % 4] + v * v
            return tuple(a)

        return lax.fori_loop(0, _CH // 4, rows_body, l2accs)

    for t in range(3):
        issue(t)
    l2accs = tuple(jnp.zeros((16,), jnp.float32) for _ in range(4))
    for t in range(ntask):
        if t + 3 < ntask:
            issue(t + 3)
        for h in handles[t]:
            h.wait()
        if t < _NCHUNK:
            compute_cur(t)
        else:
            l2accs = compute_raw(t, l2accs)

    pltpu.sync_copy(dv, out_diff.at[pl.ds(row0, _NCHUNK)])
    l2v[...] = (l2accs[0] + l2accs[1]) + (l2accs[2] + l2accs[3])
    pltpu.sync_copy(l2v, out_l2.at[wid])


def _combine_body(diff_ref, l2_ref, out_ref):
    x = diff_ref[...]
    sp = jnp.maximum(x, 0.0) + jnp.log1p(jnp.exp(-jnp.abs(x)))
    out_ref[0, 0] = (jnp.sum(sp) / _B
                     + (_WEIGHT_DECAY * 0.5 / _B) * jnp.sum(l2_ref[...]))


_combine = pl.pallas_call(
    _combine_body,
    out_shape=jax.ShapeDtypeStruct((1, 1), jnp.float32),
    out_specs=pl.BlockSpec(memory_space=pltpu.SMEM),
)


def kernel(emb_users, emb_items, raw_emb_users, raw_emb_items,
           users, pos_items, neg_items):
    u2 = users.reshape(128, 128)
    p2 = pos_items.reshape(128, 128)
    n2 = neg_items.reshape(128, 128)
    diff, l2 = _sc_bpr(emb_users, emb_items, raw_emb_users, raw_emb_items,
                       u2, p2, n2)
    out = _combine(diff, l2.reshape(4, 128))
    return out[0, 0]


# speedup vs baseline: 1.1550x; 1.0003x over previous
import functools

import jax
import jax.numpy as jnp
from jax import lax
from jax.experimental import pallas as pl
from jax.experimental.pallas import tpu as pltpu
from jax.experimental.pallas import tpu_sc as plsc

_WEIGHT_DECAY = 1e-4
_D = 64
_B = 16384
_NC, _NS = 2, 16
_NW = _NC * _NS
_BW = _B // _NW
_CH = 64
_NCHUNK = _BW // _CH
_NSLOT = 4

_mesh = plsc.VectorSubcoreMesh(
    core_axis_name="c", subcore_axis_name="s",
    num_cores=_NC, num_subcores=_NS)

_scratch = (
    [pltpu.VMEM((8, 128), jnp.int32) for _ in range(3)]
    + [pltpu.VMEM((_NCHUNK, _CH), jnp.int32) for _ in range(3)]
    + [pltpu.VMEM((_NCHUNK, _CH), jnp.int32) for _ in range(3)]
    + [pltpu.VMEM((_CH, 128), jnp.float32) for _ in range(3 * _NSLOT)]
    + [pltpu.VMEM((_BW,), jnp.float32),
       pltpu.VMEM((16,), jnp.float32)]
    + [pltpu.SemaphoreType.DMA for _ in range(_NSLOT)]
)


@functools.partial(
    pl.kernel,
    out_type=[
        jax.ShapeDtypeStruct((_B,), jnp.float32),
        jax.ShapeDtypeStruct((_NW * 16,), jnp.float32),
    ],
    mesh=_mesh,
    compiler_params=pltpu.CompilerParams(
        needs_layout_passes=False, use_tc_tiling_on_sc=True),
    scratch_types=_scratch,
)
def _sc_bpr(emb_u, emb_i, raw_u, raw_i, users, pos, neg,
            out_diff, out_l2, *refs):
    idx = refs[0:3]
    hidx = refs[3:6]
    par = refs[6:9]
    bufs = [refs[9 + 3 * s: 12 + 3 * s] for s in range(_NSLOT)]
    dv, l2v = refs[9 + 3 * _NSLOT], refs[10 + 3 * _NSLOT]
    sems = refs[11 + 3 * _NSLOT:]

    wid = lax.axis_index("s") * _NC + lax.axis_index("c")
    blk = wid // 2
    r0 = (wid % 2) * 4

    pltpu.sync_copy(users.at[blk], idx[0])
    pltpu.sync_copy(pos.at[blk], idx[1])
    pltpu.sync_copy(neg.at[blk], idx[2])

    iota = jnp.arange(16, dtype=jnp.int32)

    for k in range(3):
        for c in range(_NCHUNK):
            for g in range(_CH // 16):
                v = idx[k][r0 + c // 2, pl.ds((c % 2) * 64 + g * 16, 16)]
                hidx[k][c, pl.ds(g * 16, 16)] = v >> 1
                par[k][c, pl.ds(g * 16, 16)] = (v & 1) * 64

    ntask = 2 * _NCHUNK
    handles = [None] * ntask

    def issue(t):
        ph, ch = divmod(t, _NCHUNK)
        slot = t % _NSLOT
        tabs = (emb_u, emb_i, emb_i) if ph == 0 else (raw_u, raw_i, raw_i)
        handles[t] = [
            pltpu.async_copy(tabs[k].at[hidx[k].at[ch]], bufs[slot][k], sems[slot])
            for k in range(3)
        ]

    def col(pv, d):
        c = iota + d
        c = c - jnp.where(c >= _D, _D, 0).astype(jnp.int32)
        return pv + c

    def compute_cur(t):
        ch = t % _NCHUNK
        bu, bp, bn = bufs[t % _NSLOT]

        def group_body(g, _):
            i1 = g * 16 + iota
            pu = par[0][ch, pl.ds(g * 16, 16)]
            pp = par[1][ch, pl.ds(g * 16, 16)]
            pn = par[2][ch, pl.ds(g * 16, 16)]
            accs = [jnp.zeros((16,), jnp.float32) for _ in range(4)]
            for d in range(_D):
                u = plsc.load_gather(bu, [i1, col(pu, d)])
                p = plsc.load_gather(bp, [i1, col(pp, d)])
                n = plsc.load_gather(bn, [i1, col(pn, d)])
                accs[d % 4] = accs[d % 4] + u * (n - p)
            dv[pl.ds(ch * _CH + g * 16, 16)] = (accs[0] + accs[1]) + (accs[2] + accs[3])
            return 0

        lax.fori_loop(0, _CH // 16, group_body, 0)

    def compute_raw(t, l2accs):
        ch = t % _NCHUNK
        bu, bp, bn = bufs[t % _NSLOT]

        def group_body(g, accs):
            i1 = g * 16 + iota
            pu = par[0][ch, pl.ds(g * 16, 16)]
            pp = par[1][ch, pl.ds(g * 16, 16)]
            pn = par[2][ch, pl.ds(g * 16, 16)]
            a = list(accs)
            for d in range(_D):
                u = plsc.load_gather(bu, [i1, col(pu, d)])
                p = plsc.load_gather(bp, [i1, col(pp, d)])
                n = plsc.load_gather(bn, [i1, col(pn, d)])
                a[d % 4] = a[d % 4] + (u * u + (p * p + n * n))
            return tuple(a)

        return lax.fori_loop(0, _CH // 16, group_body, l2accs)

    for t in range(3):
        issue(t)
    l2accs = tuple(jnp.zeros((16,), jnp.float32) for _ in range(4))
    for t in range(ntask):
        if t + 3 < ntask:
            issue(t + 3)
        for h in handles[t]:
            h.wait()
        if t < _NCHUNK:
            compute_cur(t)
        else:
            l2accs = compute_raw(t, l2accs)

    pltpu.sync_copy(dv, out_diff.at[pl.ds(wid * _BW, _BW)])
    l2v[...] = (l2accs[0] + l2accs[1]) + (l2accs[2] + l2accs[3])
    pltpu.sync_copy(l2v, out_l2.at[pl.ds(wid * 16, 16)])


def _combine_body(diff_ref, l2_ref, out_ref):
    x = diff_ref[...]
    sp = jnp.maximum(x, 0.0) + jnp.log1p(jnp.exp(-jnp.abs(x)))
    out_ref[0, 0] = (jnp.sum(sp) / _B
                     + (_WEIGHT_DECAY * 0.5 / _B) * jnp.sum(l2_ref[...]))


_combine = pl.pallas_call(
    _combine_body,
    out_shape=jax.ShapeDtypeStruct((1, 1), jnp.float32),
    out_specs=pl.BlockSpec(memory_space=pltpu.SMEM),
)


def kernel(emb_users, emb_items, raw_emb_users, raw_emb_items,
           users, pos_items, neg_items):
    eu = emb_users.reshape(50000, 128)
    ei = emb_items.reshape(50000, 128)
    ru = raw_emb_users.reshape(50000, 128)
    ri = raw_emb_items.reshape(50000, 128)
    u3 = users.reshape(16, 8, 128)
    p3 = pos_items.reshape(16, 8, 128)
    n3 = neg_items.reshape(16, 8, 128)
    diff, l2 = _sc_bpr(eu, ei, ru, ri, u3, p3, n3)
    out = _combine(diff.reshape(128, 128), l2.reshape(4, 128))
    return out[0, 0]


# speedup vs baseline: 1.6272x; 1.4088x over previous
import functools

import jax
import jax.numpy as jnp
from jax import lax
from jax.experimental import pallas as pl
from jax.experimental.pallas import tpu as pltpu
from jax.experimental.pallas import tpu_sc as plsc

_WEIGHT_DECAY = 1e-4
_D = 64
_B = 16384
_NV = 100000
_NC, _NS = 2, 16
_NW = _NC * _NS
_BW = _B // _NW
_CH = 128
_NCHUNK = _BW // _CH
_NSLOT = 3
_NBLK = 24

_mesh = plsc.VectorSubcoreMesh(
    core_axis_name="c", subcore_axis_name="s",
    num_cores=_NC, num_subcores=_NS)


@functools.partial(
    pl.kernel,
    out_type=[
        jax.ShapeDtypeStruct((_NV,), jnp.float32),
        jax.ShapeDtypeStruct((_NV,), jnp.float32),
    ],
    mesh=_mesh,
    compiler_params=pltpu.CompilerParams(
        needs_layout_passes=False, use_tc_tiling_on_sc=True),
    scratch_types=[
        pltpu.VMEM((_D, 128), jnp.float32),
        pltpu.VMEM((_D, 32), jnp.float32),
        pltpu.VMEM((128,), jnp.float32),
    ],
)
def _sc_norms(rawT_u, rawT_i, nu, ni, blk, tailb, nstage):
    wid = lax.axis_index("s") * _NC + lax.axis_index("c")

    def do_block(tab, out, coff, width, buf):
        pltpu.sync_copy(tab.at[:, pl.ds(coff, width)], buf)

        def feat_body(f, accs):
            out_accs = []
            for g in range(width // 16):
                v = buf[f, pl.ds(g * 16, 16)]
                out_accs.append(accs[g] + v * v)
            return tuple(out_accs)

        accs = lax.fori_loop(
            0, _D, feat_body,
            tuple(jnp.zeros((16,), jnp.float32) for _ in range(width // 16)))
        for g in range(width // 16):
            nstage[pl.ds(g * 16, 16)] = accs[g]
        pltpu.sync_copy(nstage.at[pl.ds(0, width)], out.at[pl.ds(coff, width)])

    for tab, out in ((rawT_u, nu), (rawT_i, ni)):
        def blk_body(j, _, tab=tab, out=out):
            do_block(tab, out, (wid + _NW * j) * 128, 128, blk)
            return 0

        lax.fori_loop(0, _NBLK, blk_body, 0)
        @pl.when(wid <= 12)
        def _():
            do_block(tab, out, (_NW * _NBLK + wid) * 128, 128, blk)
        @pl.when(wid == 31)
        def _():
            do_block(tab, out, 99968, 32, tailb)


_scratch = (
    [pltpu.VMEM((_NCHUNK, _CH), jnp.int32) for _ in range(3)]
    + [pltpu.VMEM((_NCHUNK, _CH), jnp.int32) for _ in range(3)]
    + [pltpu.VMEM((_NCHUNK, _CH), jnp.int32) for _ in range(3)]
    + [pltpu.VMEM((_CH, _D), jnp.float32) for _ in range(3 * _NSLOT)]
    + [pltpu.VMEM((_CH, 8), jnp.float32) for _ in range(3 * _NCHUNK)]
    + [pltpu.VMEM((_NCHUNK, _CH), jnp.float32),
       pltpu.VMEM((16,), jnp.float32)]
    + [pltpu.SemaphoreType.DMA for _ in range(_NSLOT)]
    + [pltpu.SemaphoreType.DMA]
)


@functools.partial(
    pl.kernel,
    out_type=[
        jax.ShapeDtypeStruct((128, 128), jnp.float32),
        jax.ShapeDtypeStruct((_NW, 16), jnp.float32),
    ],
    mesh=_mesh,
    compiler_params=pltpu.CompilerParams(
        needs_layout_passes=False, use_tc_tiling_on_sc=False),
    scratch_types=_scratch,
)
def _sc_main(emb_u, emb_i, nu8, ni8, users, pos, neg,
             out_diff, out_l2, *refs):
    idx = refs[0:3]
    hidx = refs[3:6]
    lidx = refs[6:9]
    bufs = [refs[9 + 3 * s: 12 + 3 * s] for s in range(_NSLOT)]
    _o = 9 + 3 * _NSLOT
    nbufs = [refs[_o + 3 * c: _o + 3 * c + 3] for c in range(_NCHUNK)]
    _o += 3 * _NCHUNK
    dv, l2v = refs[_o], refs[_o + 1]
    sems = refs[_o + 2: _o + 2 + _NSLOT]
    nsem = refs[_o + 2 + _NSLOT]

    wid = lax.axis_index("s") * _NC + lax.axis_index("c")
    row0 = wid * _NCHUNK

    pltpu.sync_copy(users.at[pl.ds(row0, _NCHUNK)], idx[0])
    pltpu.sync_copy(pos.at[pl.ds(row0, _NCHUNK)], idx[1])
    pltpu.sync_copy(neg.at[pl.ds(row0, _NCHUNK)], idx[2])

    iota = jnp.arange(16, dtype=jnp.int32)

    for k in range(3):
        for c in range(_NCHUNK):
            for g in range(_CH // 16):
                v = idx[k][c, pl.ds(g * 16, 16)]
                hidx[k][c, pl.ds(g * 16, 16)] = v >> 3
                lidx[k][c, pl.ds(g * 16, 16)] = v & 7

    ntabs = (nu8, ni8, ni8)
    nhandles = [
        pltpu.async_copy(ntabs[k].at[hidx[k].at[c]], nbufs[c][k], nsem)
        for c in range(_NCHUNK) for k in range(3)
    ]

    handles = [None] * _NCHUNK

    def issue(t):
        slot = t % _NSLOT
        tabs = (emb_u, emb_i, emb_i)
        handles[t] = [
            pltpu.async_copy(tabs[k].at[idx[k].at[t]], bufs[slot][k], sems[slot])
            for k in range(3)
        ]

    def compute_cur(t):
        bu, bp, bn = bufs[t % _NSLOT]

        def group_body(g, _):
            i1 = g * 16 + iota
            accs = [jnp.zeros((16,), jnp.float32) for _ in range(4)]
            for d in range(_D):
                i2 = iota + d
                i2 = i2 - jnp.where(i2 >= _D, _D, 0).astype(jnp.int32)
                u = plsc.load_gather(bu, [i1, i2])
                p = plsc.load_gather(bp, [i1, i2])
                n = plsc.load_gather(bn, [i1, i2])
                accs[d % 4] = accs[d % 4] + u * (n - p)
            dv[t, pl.ds(g * 16, 16)] = (accs[0] + accs[1]) + (accs[2] + accs[3])
            return 0

        lax.fori_loop(0, _CH // 16, group_body, 0)

    for t in range(2):
        issue(t)
    for t in range(_NCHUNK):
        if t + 2 < _NCHUNK:
            issue(t + 2)
        for h in handles[t]:
            h.wait()
        compute_cur(t)

    pltpu.sync_copy(dv, out_diff.at[pl.ds(row0, _NCHUNK)])

    for h in nhandles:
        h.wait()
    accs = [jnp.zeros((16,), jnp.float32) for _ in range(4)]
    for c in range(_NCHUNK):
        for g in range(_CH // 16):
            i1 = g * 16 + iota
            for k in range(3):
                i2 = lidx[k][c, pl.ds(g * 16, 16)]
                accs[(g + k) % 4] = accs[(g + k) % 4] + plsc.load_gather(
                    nbufs[c][k], [i1, i2])
    l2v[...] = (accs[0] + accs[1]) + (accs[2] + accs[3])
    pltpu.sync_copy(l2v, out_l2.at[wid])


def _combine_body(diff_ref, l2_ref, out_ref):
    x = diff_ref[...]
    sp = jnp.maximum(x, 0.0) + jnp.log1p(jnp.exp(-jnp.abs(x)))
    out_ref[0, 0] = (jnp.sum(sp) / _B
                     + (_WEIGHT_DECAY * 0.5 / _B) * jnp.sum(l2_ref[...]))


_combine = pl.pallas_call(
    _combine_body,
    out_shape=jax.ShapeDtypeStruct((1, 1), jnp.float32),
    out_specs=pl.BlockSpec(memory_space=pltpu.SMEM),
)


def kernel(emb_users, emb_items, raw_emb_users, raw_emb_items,
           users, pos_items, neg_items):
    nu, ni = _sc_norms(raw_emb_users.T, raw_emb_items.T)
    u2 = users.reshape(128, 128)
    p2 = pos_items.reshape(128, 128)
    n2 = neg_items.reshape(128, 128)
    diff, l2 = _sc_main(emb_users, emb_items,
                        nu.reshape(12500, 8), ni.reshape(12500, 8),
                        u2, p2, n2)
    out = _combine(diff, l2.reshape(4, 128))
    return out[0, 0]


# speedup vs baseline: 1.8470x; 1.1351x over previous
import functools

import jax
import jax.numpy as jnp
from jax import lax
from jax.experimental import pallas as pl
from jax.experimental.pallas import tpu as pltpu
from jax.experimental.pallas import tpu_sc as plsc

_WEIGHT_DECAY = 1e-4
_D = 64
_B = 16384
_NV = 100000
_NC, _NS = 2, 16
_NW = _NC * _NS
_BW = _B // _NW
_CH = 128
_NCHUNK = _BW // _CH
_NSLOT = 3
_NBLK = 24

_mesh = plsc.VectorSubcoreMesh(
    core_axis_name="c", subcore_axis_name="s",
    num_cores=_NC, num_subcores=_NS)


@functools.partial(
    pl.kernel,
    out_type=[
        jax.ShapeDtypeStruct((_NV,), jnp.float32),
        jax.ShapeDtypeStruct((_NV,), jnp.float32),
    ],
    mesh=_mesh,
    compiler_params=pltpu.CompilerParams(
        needs_layout_passes=False, use_tc_tiling_on_sc=True),
    scratch_types=[
        pltpu.VMEM((_D, 128), jnp.float32),
        pltpu.VMEM((_D, 128), jnp.float32),
        pltpu.VMEM((_D, 32), jnp.float32),
        pltpu.VMEM((128,), jnp.float32),
        pltpu.SemaphoreType.DMA,
        pltpu.SemaphoreType.DMA,
    ],
)
def _sc_norms(rawT_u, rawT_i, nu, ni, blk0, blk1, tailb, nstage, sem0, sem1):
    wid = lax.axis_index("s") * _NC + lax.axis_index("c")
    blks = (blk0, blk1)
    sems = (sem0, sem1)

    def compute_block(buf, out, coff, width):
        def feat_body(f, accs):
            out_accs = []
            for g in range(width // 16):
                v = buf[f, pl.ds(g * 16, 16)]
                out_accs.append(accs[g] + v * v)
            return tuple(out_accs)

        accs = lax.fori_loop(
            0, _D, feat_body,
            tuple(jnp.zeros((16,), jnp.float32) for _ in range(width // 16)))
        for g in range(width // 16):
            nstage[pl.ds(g * 16, 16)] = accs[g]
        pltpu.sync_copy(nstage.at[pl.ds(0, width)], out.at[pl.ds(coff, width)])

    work = ([(rawT_u, nu, j) for j in range(_NBLK)]
            + [(rawT_i, ni, j) for j in range(_NBLK)])
    handles = [None] * len(work)

    def issue(t):
        tab, _, j = work[t]
        handles[t] = pltpu.async_copy(
            tab.at[:, pl.ds((wid + _NW * j) * 128, 128)], blks[t % 2], sems[t % 2])

    issue(0)
    issue(1)
    for t in range(len(work)):
        handles[t].wait()
        _, out, j = work[t]
        compute_block(blks[t % 2], out, (wid + _NW * j) * 128, 128)
        if t + 2 < len(work):
            issue(t + 2)

    for tab, out in ((rawT_u, nu), (rawT_i, ni)):
        @pl.when(wid <= 12)
        def _(tab=tab, out=out):
            coff = (_NW * _NBLK + wid) * 128
            pltpu.sync_copy(tab.at[:, pl.ds(coff, 128)], blk0)
            compute_block(blk0, out, coff, 128)
        @pl.when(wid == 31)
        def _(tab=tab, out=out):
            pltpu.sync_copy(tab.at[:, pl.ds(99968, 32)], tailb)
            compute_block(tailb, out, 99968, 32)


_scratch = (
    [pltpu.VMEM((_NCHUNK, _CH), jnp.int32) for _ in range(3)]
    + [pltpu.VMEM((_NCHUNK, _CH), jnp.int32) for _ in range(3)]
    + [pltpu.VMEM((_NCHUNK, _CH), jnp.int32) for _ in range(3)]
    + [pltpu.VMEM((_CH, _D), jnp.float32) for _ in range(3 * _NSLOT)]
    + [pltpu.VMEM((_CH, 8), jnp.float32) for _ in range(3 * _NCHUNK)]
    + [pltpu.VMEM((_NCHUNK, _CH), jnp.float32),
       pltpu.VMEM((16,), jnp.float32)]
    + [pltpu.SemaphoreType.DMA for _ in range(_NSLOT)]
    + [pltpu.SemaphoreType.DMA]
)


@functools.partial(
    pl.kernel,
    out_type=[
        jax.ShapeDtypeStruct((128, 128), jnp.float32),
        jax.ShapeDtypeStruct((_NW, 16), jnp.float32),
    ],
    mesh=_mesh,
    compiler_params=pltpu.CompilerParams(
        needs_layout_passes=False, use_tc_tiling_on_sc=False),
    scratch_types=_scratch,
)
def _sc_main(emb_u, emb_i, nu8, ni8, users, pos, neg,
             out_diff, out_l2, *refs):
    idx = refs[0:3]
    hidx = refs[3:6]
    lidx = refs[6:9]
    bufs = [refs[9 + 3 * s: 12 + 3 * s] for s in range(_NSLOT)]
    _o = 9 + 3 * _NSLOT
    nbufs = [refs[_o + 3 * c: _o + 3 * c + 3] for c in range(_NCHUNK)]
    _o += 3 * _NCHUNK
    dv, l2v = refs[_o], refs[_o + 1]
    sems = refs[_o + 2: _o + 2 + _NSLOT]
    nsem = refs[_o + 2 + _NSLOT]

    wid = lax.axis_index("s") * _NC + lax.axis_index("c")
    row0 = wid * _NCHUNK

    pltpu.sync_copy(users.at[pl.ds(row0, _NCHUNK)], idx[0])
    pltpu.sync_copy(pos.at[pl.ds(row0, _NCHUNK)], idx[1])
    pltpu.sync_copy(neg.at[pl.ds(row0, _NCHUNK)], idx[2])

    iota = jnp.arange(16, dtype=jnp.int32)

    for k in range(3):
        for c in range(_NCHUNK):
            for g in range(_CH // 16):
                v = idx[k][c, pl.ds(g * 16, 16)]
                hidx[k][c, pl.ds(g * 16, 16)] = v >> 3
                lidx[k][c, pl.ds(g * 16, 16)] = v & 7

    ntabs = (nu8, ni8, ni8)
    nhandles = [
        pltpu.async_copy(ntabs[k].at[hidx[k].at[c]], nbufs[c][k], nsem)
        for c in range(_NCHUNK) for k in range(3)
    ]

    handles = [None] * _NCHUNK

    def issue(t):
        slot = t % _NSLOT
        tabs = (emb_u, emb_i, emb_i)
        handles[t] = [
            pltpu.async_copy(tabs[k].at[idx[k].at[t]], bufs[slot][k], sems[slot])
            for k in range(3)
        ]

    def compute_cur(t):
        bu, bp, bn = bufs[t % _NSLOT]

        def group_body(g, _):
            i1 = g * 16 + iota
            accs = [jnp.zeros((16,), jnp.float32) for _ in range(4)]
            for d in range(_D):
                i2 = iota + d
                i2 = i2 - jnp.where(i2 >= _D, _D, 0).astype(jnp.int32)
                u = plsc.load_gather(bu, [i1, i2])
                p = plsc.load_gather(bp, [i1, i2])
                n = plsc.load_gather(bn, [i1, i2])
                accs[d % 4] = accs[d % 4] + u * (n - p)
            dv[t, pl.ds(g * 16, 16)] = (accs[0] + accs[1]) + (accs[2] + accs[3])
            return 0

        lax.fori_loop(0, _CH // 16, group_body, 0)

    for t in range(2):
        issue(t)
    for t in range(_NCHUNK):
        if t + 2 < _NCHUNK:
            issue(t + 2)
        for h in handles[t]:
            h.wait()
        compute_cur(t)

    pltpu.sync_copy(dv, out_diff.at[pl.ds(row0, _NCHUNK)])

    for h in nhandles:
        h.wait()
    accs = [jnp.zeros((16,), jnp.float32) for _ in range(4)]
    for c in range(_NCHUNK):
        for g in range(_CH // 16):
            i1 = g * 16 + iota
            for k in range(3):
                i2 = lidx[k][c, pl.ds(g * 16, 16)]
                accs[(g + k) % 4] = accs[(g + k) % 4] + plsc.load_gather(
                    nbufs[c][k], [i1, i2])
    l2v[...] = (accs[0] + accs[1]) + (accs[2] + accs[3])
    pltpu.sync_copy(l2v, out_l2.at[wid])


def _combine_body(diff_ref, l2_ref, out_ref):
    x = diff_ref[...]
    sp = jnp.maximum(x, 0.0) + jnp.log1p(jnp.exp(-jnp.abs(x)))
    out_ref[0, 0] = (jnp.sum(sp) / _B
                     + (_WEIGHT_DECAY * 0.5 / _B) * jnp.sum(l2_ref[...]))


_combine = pl.pallas_call(
    _combine_body,
    out_shape=jax.ShapeDtypeStruct((1, 1), jnp.float32),
    out_specs=pl.BlockSpec(memory_space=pltpu.SMEM),
)


def kernel(emb_users, emb_items, raw_emb_users, raw_emb_items,
           users, pos_items, neg_items):
    nu, ni = _sc_norms(raw_emb_users.T, raw_emb_items.T)
    u2 = users.reshape(128, 128)
    p2 = pos_items.reshape(128, 128)
    n2 = neg_items.reshape(128, 128)
    diff, l2 = _sc_main(emb_users, emb_items,
                        nu.reshape(12500, 8), ni.reshape(12500, 8),
                        u2, p2, n2)
    out = _combine(diff, l2.reshape(4, 128))
    return out[0, 0]


# speedup vs baseline: 1.8507x; 1.0020x over previous
import functools

import jax
import jax.numpy as jnp
from jax import lax
from jax.experimental import pallas as pl
from jax.experimental.pallas import tpu as pltpu
from jax.experimental.pallas import tpu_sc as plsc

_WEIGHT_DECAY = 1e-4
_D = 64
_B = 16384
_NV = 100000
_NC, _NS = 2, 16
_NW = _NC * _NS
_BW = _B // _NW
_CH = 128
_NCHUNK = _BW // _CH
_NSLOT = 3
_NBLK = 24

_mesh = plsc.VectorSubcoreMesh(
    core_axis_name="c", subcore_axis_name="s",
    num_cores=_NC, num_subcores=_NS)


@functools.partial(
    pl.kernel,
    out_type=[
        jax.ShapeDtypeStruct((_NV,), jnp.float32),
        jax.ShapeDtypeStruct((_NV,), jnp.float32),
    ],
    mesh=_mesh,
    compiler_params=pltpu.CompilerParams(
        needs_layout_passes=False, use_tc_tiling_on_sc=True),
    scratch_types=[
        pltpu.VMEM((_D, 128), jnp.float32),
        pltpu.VMEM((_D, 128), jnp.float32),
        pltpu.VMEM((_D, 32), jnp.float32),
        pltpu.VMEM((4, 128), jnp.float32),
        pltpu.SemaphoreType.DMA,
        pltpu.SemaphoreType.DMA,
        pltpu.SemaphoreType.DMA,
    ],
)
def _sc_norms(rawT_u, rawT_i, nu, ni, blk0, blk1, tailb, nstage,
              sem0, sem1, wsem):
    wid = lax.axis_index("s") * _NC + lax.axis_index("c")
    blks = (blk0, blk1)
    sems = (sem0, sem1)

    def reduce_block(buf, width):
        def feat_body(f, accs):
            out_accs = []
            for g in range(width // 16):
                v = buf[f, pl.ds(g * 16, 16)]
                out_accs.append(accs[g] + v * v)
            return tuple(out_accs)

        return lax.fori_loop(
            0, _D, feat_body,
            tuple(jnp.zeros((16,), jnp.float32) for _ in range(width // 16)))

    work = ([(rawT_u, nu, j) for j in range(_NBLK)]
            + [(rawT_i, ni, j) for j in range(_NBLK)])
    handles = [None] * len(work)
    whandles = [None] * len(work)

    def issue(t):
        tab, _, j = work[t]
        handles[t] = pltpu.async_copy(
            tab.at[:, pl.ds((wid + _NW * j) * 128, 128)], blks[t % 2], sems[t % 2])

    issue(0)
    issue(1)
    for t in range(len(work)):
        handles[t].wait()
        _, out, j = work[t]
        accs = reduce_block(blks[t % 2], 128)
        if t >= 4:
            whandles[t - 4].wait()
        for g in range(8):
            nstage[t % 4, pl.ds(g * 16, 16)] = accs[g]
        whandles[t] = pltpu.async_copy(
            nstage.at[t % 4], out.at[pl.ds((wid + _NW * j) * 128, 128)], wsem)
        if t + 2 < len(work):
            issue(t + 2)
    for t in range(len(work) - 4, len(work)):
        whandles[t].wait()

    for tab, out in ((rawT_u, nu), (rawT_i, ni)):
        @pl.when(wid <= 12)
        def _(tab=tab, out=out):
            coff = (_NW * _NBLK + wid) * 128
            pltpu.sync_copy(tab.at[:, pl.ds(coff, 128)], blk0)
            accs = reduce_block(blk0, 128)
            for g in range(8):
                nstage[0, pl.ds(g * 16, 16)] = accs[g]
            pltpu.sync_copy(nstage.at[0], out.at[pl.ds(coff, 128)])
        @pl.when(wid == 31)
        def _(tab=tab, out=out):
            pltpu.sync_copy(tab.at[:, pl.ds(99968, 32)], tailb)
            accs = reduce_block(tailb, 32)
            for g in range(2):
                nstage[0, pl.ds(g * 16, 16)] = accs[g]
            pltpu.sync_copy(nstage.at[0, pl.ds(0, 32)], out.at[pl.ds(99968, 32)])


_scratch = (
    [pltpu.VMEM((_NCHUNK, _CH), jnp.int32) for _ in range(3)]
    + [pltpu.VMEM((_NCHUNK, _CH), jnp.int32) for _ in range(3)]
    + [pltpu.VMEM((_NCHUNK, _CH), jnp.int32) for _ in range(3)]
    + [pltpu.VMEM((_CH, _D), jnp.float32) for _ in range(3 * _NSLOT)]
    + [pltpu.VMEM((_CH, 8), jnp.float32) for _ in range(3 * _NCHUNK)]
    + [pltpu.VMEM((_NCHUNK, _CH), jnp.float32),
       pltpu.VMEM((16,), jnp.float32)]
    + [pltpu.SemaphoreType.DMA for _ in range(_NSLOT)]
    + [pltpu.SemaphoreType.DMA]
)


@functools.partial(
    pl.kernel,
    out_type=[
        jax.ShapeDtypeStruct((128, 128), jnp.float32),
        jax.ShapeDtypeStruct((_NW, 16), jnp.float32),
    ],
    mesh=_mesh,
    compiler_params=pltpu.CompilerParams(
        needs_layout_passes=False, use_tc_tiling_on_sc=False),
    scratch_types=_scratch,
)
def _sc_main(emb_u, emb_i, nu8, ni8, users, pos, neg,
             out_diff, out_l2, *refs):
    idx = refs[0:3]
    hidx = refs[3:6]
    lidx = refs[6:9]
    bufs = [refs[9 + 3 * s: 12 + 3 * s] for s in range(_NSLOT)]
    _o = 9 + 3 * _NSLOT
    nbufs = [refs[_o + 3 * c: _o + 3 * c + 3] for c in range(_NCHUNK)]
    _o += 3 * _NCHUNK
    dv, l2v = refs[_o], refs[_o + 1]
    sems = refs[_o + 2: _o + 2 + _NSLOT]
    nsem = refs[_o + 2 + _NSLOT]

    wid = lax.axis_index("s") * _NC + lax.axis_index("c")
    row0 = wid * _NCHUNK

    pltpu.sync_copy(users.at[pl.ds(row0, _NCHUNK)], idx[0])
    pltpu.sync_copy(pos.at[pl.ds(row0, _NCHUNK)], idx[1])
    pltpu.sync_copy(neg.at[pl.ds(row0, _NCHUNK)], idx[2])

    iota = jnp.arange(16, dtype=jnp.int32)

    for k in range(3):
        for c in range(_NCHUNK):
            for g in range(_CH // 16):
                v = idx[k][c, pl.ds(g * 16, 16)]
                hidx[k][c, pl.ds(g * 16, 16)] = v >> 3
                lidx[k][c, pl.ds(g * 16, 16)] = v & 7

    ntabs = (nu8, ni8, ni8)
    nhandles = [
        pltpu.async_copy(ntabs[k].at[hidx[k].at[c]], nbufs[c][k], nsem)
        for c in range(_NCHUNK) for k in range(3)
    ]

    handles = [None] * _NCHUNK

    def issue(t):
        slot = t % _NSLOT
        tabs = (emb_u, emb_i, emb_i)
        handles[t] = [
            pltpu.async_copy(tabs[k].at[idx[k].at[t]], bufs[slot][k], sems[slot])
            for k in range(3)
        ]

    def compute_cur(t):
        bu, bp, bn = bufs[t % _NSLOT]

        def group_body(g, _):
            i1 = g * 16 + iota
            accs = [jnp.zeros((16,), jnp.float32) for _ in range(4)]
            for d in range(_D):
                i2 = iota + d
                i2 = i2 - jnp.where(i2 >= _D, _D, 0).astype(jnp.int32)
                u = plsc.load_gather(bu, [i1, i2])
                p = plsc.load_gather(bp, [i1, i2])
                n = plsc.load_gather(bn, [i1, i2])
                accs[d % 4] = accs[d % 4] + u * (n - p)
            dv[t, pl.ds(g * 16, 16)] = (accs[0] + accs[1]) + (accs[2] + accs[3])
            return 0

        lax.fori_loop(0, _CH // 16, group_body, 0)

    for t in range(2):
        issue(t)
    for t in range(_NCHUNK):
        if t + 2 < _NCHUNK:
            issue(t + 2)
        for h in handles[t]:
            h.wait()
        compute_cur(t)

    pltpu.sync_copy(dv, out_diff.at[pl.ds(row0, _NCHUNK)])

    for h in nhandles:
        h.wait()
    accs = [jnp.zeros((16,), jnp.float32) for _ in range(4)]
    for c in range(_NCHUNK):
        for g in range(_CH // 16):
            i1 = g * 16 + iota
            for k in range(3):
                i2 = lidx[k][c, pl.ds(g * 16, 16)]
                accs[(g + k) % 4] = accs[(g + k) % 4] + plsc.load_gather(
                    nbufs[c][k], [i1, i2])
    l2v[...] = (accs[0] + accs[1]) + (accs[2] + accs[3])
    pltpu.sync_copy(l2v, out_l2.at[wid])


def _combine_body(diff_ref, l2_ref, out_ref):
    x = diff_ref[...]
    sp = jnp.maximum(x, 0.0) + jnp.log1p(jnp.exp(-jnp.abs(x)))
    out_ref[0, 0] = (jnp.sum(sp) / _B
                     + (_WEIGHT_DECAY * 0.5 / _B) * jnp.sum(l2_ref[...]))


_combine = pl.pallas_call(
    _combine_body,
    out_shape=jax.ShapeDtypeStruct((1, 1), jnp.float32),
    out_specs=pl.BlockSpec(memory_space=pltpu.SMEM),
)


def kernel(emb_users, emb_items, raw_emb_users, raw_emb_items,
           users, pos_items, neg_items):
    nu, ni = _sc_norms(raw_emb_users.T, raw_emb_items.T)
    u2 = users.reshape(128, 128)
    p2 = pos_items.reshape(128, 128)
    n2 = neg_items.reshape(128, 128)
    diff, l2 = _sc_main(emb_users, emb_items,
                        nu.reshape(12500, 8), ni.reshape(12500, 8),
                        u2, p2, n2)
    out = _combine(diff, l2.reshape(4, 128))
    return out[0, 0]


# speedup vs baseline: 2.0957x; 1.1324x over previous
import functools

import jax
import jax.numpy as jnp
from jax import lax
from jax.experimental import pallas as pl
from jax.experimental.pallas import tpu as pltpu
from jax.experimental.pallas import tpu_sc as plsc

_WEIGHT_DECAY = 1e-4
_D = 64
_B = 16384
_NV = 100000
_NC, _NS = 2, 16
_NW = _NC * _NS
_BW = _B // _NW
_CH = 128
_NCHUNK = _BW // _CH
_NSLOT = 3
_NBLK = 24

_mesh = plsc.VectorSubcoreMesh(
    core_axis_name="c", subcore_axis_name="s",
    num_cores=_NC, num_subcores=_NS)


@functools.partial(
    pl.kernel,
    out_type=[
        jax.ShapeDtypeStruct((_NV,), jnp.float32),
        jax.ShapeDtypeStruct((_NV,), jnp.float32),
    ],
    mesh=_mesh,
    compiler_params=pltpu.CompilerParams(
        needs_layout_passes=False, use_tc_tiling_on_sc=True),
    scratch_types=[
        pltpu.VMEM((_D, 128), jnp.float32),
        pltpu.VMEM((_D, 128), jnp.float32),
        pltpu.VMEM((_D, 32), jnp.float32),
        pltpu.VMEM((4, 128), jnp.float32),
        pltpu.SemaphoreType.DMA,
        pltpu.SemaphoreType.DMA,
        pltpu.SemaphoreType.DMA,
    ],
)
def _sc_norms(rawT_u, rawT_i, nu, ni, blk0, blk1, tailb, nstage,
              sem0, sem1, wsem):
    wid = lax.axis_index("s") * _NC + lax.axis_index("c")
    blks = (blk0, blk1)
    sems = (sem0, sem1)

    def reduce_block(buf, width):
        def feat_body(f, accs):
            out_accs = []
            for g in range(width // 16):
                v = buf[f, pl.ds(g * 16, 16)]
                out_accs.append(accs[g] + v * v)
            return tuple(out_accs)

        return lax.fori_loop(
            0, _D, feat_body,
            tuple(jnp.zeros((16,), jnp.float32) for _ in range(width // 16)))

    work = ([(rawT_u, nu, j) for j in range(_NBLK)]
            + [(rawT_i, ni, j) for j in range(_NBLK)])
    handles = [None] * len(work)
    whandles = [None] * len(work)

    def issue(t):
        tab, _, j = work[t]
        handles[t] = pltpu.async_copy(
            tab.at[:, pl.ds((wid + _NW * j) * 128, 128)], blks[t % 2], sems[t % 2])

    issue(0)
    issue(1)
    for t in range(len(work)):
        handles[t].wait()
        _, out, j = work[t]
        accs = reduce_block(blks[t % 2], 128)
        if t >= 4:
            whandles[t - 4].wait()
        for g in range(8):
            nstage[t % 4, pl.ds(g * 16, 16)] = accs[g]
        whandles[t] = pltpu.async_copy(
            nstage.at[t % 4], out.at[pl.ds((wid + _NW * j) * 128, 128)], wsem)
        if t + 2 < len(work):
            issue(t + 2)
    for t in range(len(work) - 4, len(work)):
        whandles[t].wait()

    for tab, out in ((rawT_u, nu), (rawT_i, ni)):
        @pl.when(wid <= 12)
        def _(tab=tab, out=out):
            coff = (_NW * _NBLK + wid) * 128
            pltpu.sync_copy(tab.at[:, pl.ds(coff, 128)], blk0)
            accs = reduce_block(blk0, 128)
            for g in range(8):
                nstage[0, pl.ds(g * 16, 16)] = accs[g]
            pltpu.sync_copy(nstage.at[0], out.at[pl.ds(coff, 128)])
        @pl.when(wid == 31)
        def _(tab=tab, out=out):
            pltpu.sync_copy(tab.at[:, pl.ds(99968, 32)], tailb)
            accs = reduce_block(tailb, 32)
            for g in range(2):
                nstage[0, pl.ds(g * 16, 16)] = accs[g]
            pltpu.sync_copy(nstage.at[0, pl.ds(0, 32)], out.at[pl.ds(99968, 32)])


_scratch = (
    [pltpu.VMEM((_NCHUNK, _CH), jnp.int32) for _ in range(3)]
    + [pltpu.VMEM((_NCHUNK, _CH), jnp.int32) for _ in range(3)]
    + [pltpu.VMEM((_NCHUNK, _CH), jnp.int32) for _ in range(3)]
    + [pltpu.VMEM((_CH, _D), jnp.float32) for _ in range(3 * _NSLOT)]
    + [pltpu.VMEM((_CH, 8), jnp.float32) for _ in range(3 * _NCHUNK)]
    + [pltpu.VMEM((_NCHUNK, _CH), jnp.float32),
       pltpu.VMEM((16,), jnp.float32)]
    + [pltpu.SemaphoreType.DMA for _ in range(_NSLOT)]
    + [pltpu.SemaphoreType.DMA]
)


@functools.partial(
    pl.kernel,
    out_type=[
        jax.ShapeDtypeStruct((128, 128), jnp.float32),
        jax.ShapeDtypeStruct((_NW, 16), jnp.float32),
    ],
    mesh=_mesh,
    compiler_params=pltpu.CompilerParams(
        needs_layout_passes=False, use_tc_tiling_on_sc=False),
    scratch_types=_scratch,
)
def _sc_main(emb_u, emb_i, nu8, ni8, users, pos, neg,
             out_diff, out_l2, *refs):
    idx = refs[0:3]
    hidx = refs[3:6]
    lidx = refs[6:9]
    bufs = [refs[9 + 3 * s: 12 + 3 * s] for s in range(_NSLOT)]
    _o = 9 + 3 * _NSLOT
    nbufs = [refs[_o + 3 * c: _o + 3 * c + 3] for c in range(_NCHUNK)]
    _o += 3 * _NCHUNK
    dv, l2v = refs[_o], refs[_o + 1]
    sems = refs[_o + 2: _o + 2 + _NSLOT]
    nsem = refs[_o + 2 + _NSLOT]

    wid = lax.axis_index("s") * _NC + lax.axis_index("c")
    row0 = wid * _NCHUNK

    pltpu.sync_copy(users.at[pl.ds(row0, _NCHUNK)], idx[0])
    pltpu.sync_copy(pos.at[pl.ds(row0, _NCHUNK)], idx[1])
    pltpu.sync_copy(neg.at[pl.ds(row0, _NCHUNK)], idx[2])

    iota = jnp.arange(16, dtype=jnp.int32)

    for k in range(3):
        for c in range(_NCHUNK):
            for g in range(_CH // 16):
                v = idx[k][c, pl.ds(g * 16, 16)]
                hidx[k][c, pl.ds(g * 16, 16)] = v >> 3
                lidx[k][c, pl.ds(g * 16, 16)] = v & 7

    ntabs = (nu8, ni8, ni8)
    nhandles = [
        pltpu.async_copy(ntabs[k].at[hidx[k].at[c]], nbufs[c][k], nsem)
        for c in range(_NCHUNK) for k in range(3)
    ]

    handles = [None] * _NCHUNK

    def issue(t):
        slot = t % _NSLOT
        tabs = (emb_u, emb_i, emb_i)
        handles[t] = [
            pltpu.async_copy(tabs[k].at[idx[k].at[t]], bufs[slot][k], sems[slot])
            for k in range(3)
        ]

    def compute_cur(t):
        bu, bp, bn = bufs[t % _NSLOT]

        def group_body(g, _):
            i1 = g * 16 + iota
            def dbody(dq, accs):
                a = list(accs)
                for q in range(4):
                    i2 = iota + (dq * 4 + q)
                    i2 = i2 - jnp.where(i2 >= _D, _D, 0).astype(jnp.int32)
                    u = plsc.load_gather(bu, [i1, i2])
                    p = plsc.load_gather(bp, [i1, i2])
                    n = plsc.load_gather(bn, [i1, i2])
                    a[q] = a[q] + u * (n - p)
                return tuple(a)

            accs = lax.fori_loop(
                0, _D // 4, dbody,
                tuple(jnp.zeros((16,), jnp.float32) for _ in range(4)))
            dv[t, pl.ds(g * 16, 16)] = (accs[0] + accs[1]) + (accs[2] + accs[3])
            return 0

        lax.fori_loop(0, _CH // 16, group_body, 0)

    for t in range(2):
        issue(t)
    for t in range(_NCHUNK):
        if t + 2 < _NCHUNK:
            issue(t + 2)
        for h in handles[t]:
            h.wait()
        compute_cur(t)

    pltpu.sync_copy(dv, out_diff.at[pl.ds(row0, _NCHUNK)])

    for h in nhandles:
        h.wait()
    accs = [jnp.zeros((16,), jnp.float32) for _ in range(4)]
    for c in range(_NCHUNK):
        for g in range(_CH // 16):
            i1 = g * 16 + iota
            for k in range(3):
                i2 = lidx[k][c, pl.ds(g * 16, 16)]
                accs[(g + k) % 4] = accs[(g + k) % 4] + plsc.load_gather(
                    nbufs[c][k], [i1, i2])
    l2v[...] = (accs[0] + accs[1]) + (accs[2] + accs[3])
    pltpu.sync_copy(l2v, out_l2.at[wid])


def _combine_body(diff_ref, l2_ref, out_ref):
    x = diff_ref[...]
    sp = jnp.maximum(x, 0.0) + jnp.log1p(jnp.exp(-jnp.abs(x)))
    out_ref[0, 0] = (jnp.sum(sp) / _B
                     + (_WEIGHT_DECAY * 0.5 / _B) * jnp.sum(l2_ref[...]))


_combine = pl.pallas_call(
    _combine_body,
    out_shape=jax.ShapeDtypeStruct((1, 1), jnp.float32),
    out_specs=pl.BlockSpec(memory_space=pltpu.SMEM),
)


def kernel(emb_users, emb_items, raw_emb_users, raw_emb_items,
           users, pos_items, neg_items):
    nu, ni = _sc_norms(raw_emb_users.T, raw_emb_items.T)
    u2 = users.reshape(128, 128)
    p2 = pos_items.reshape(128, 128)
    n2 = neg_items.reshape(128, 128)
    diff, l2 = _sc_main(emb_users, emb_items,
                        nu.reshape(12500, 8), ni.reshape(12500, 8),
                        u2, p2, n2)
    out = _combine(diff, l2.reshape(4, 128))
    return out[0, 0]
